# Initial kernel scaffold; baseline (speedup 1.0000x reference)
import jax
import jax.numpy as jnp
from jax.experimental import pallas as pl


def kernel(Y, weights, triplets):
    raise NotImplementedError("write your pallas kernel here")


# speedup vs baseline: 38.9324x; 38.9324x over previous
import functools

import jax
import jax.numpy as jnp
from jax import lax
from jax.experimental import pallas as pl
from jax.experimental.pallas import tpu as pltpu
from jax.experimental.pallas import tpu_sc as plsc

N = 100000
T = 3200000
NC, NS, L = 2, 16, 16
NW = NC * NS
TW = T // NW
B = 4000
NCHUNK = TW // B


def _tri_kernel(y0_hbm, y1_hbm, w_hbm, ti_hbm, tj_hbm, tk_hbm, out_hbm,
                idx_i, idx_j, idx_k, w_v,
                yi0, yi1, yj0, yj1, yk0, yk1, acc_v, sem):
    wid = lax.axis_index("s") * NC + lax.axis_index("c")

    def chunk_body(c, carry):
        loss_acc, viol_acc = carry
        base = wid * TW + c * B
        pltpu.sync_copy(ti_hbm.at[pl.ds(base, B)], idx_i)
        pltpu.sync_copy(tj_hbm.at[pl.ds(base, B)], idx_j)
        pltpu.sync_copy(tk_hbm.at[pl.ds(base, B)], idx_k)
        pltpu.sync_copy(w_hbm.at[pl.ds(base, B)], w_v)
        cps = [
            pltpu.async_copy(y0_hbm.at[idx_i], yi0, sem),
            pltpu.async_copy(y1_hbm.at[idx_i], yi1, sem),
            pltpu.async_copy(y0_hbm.at[idx_j], yj0, sem),
            pltpu.async_copy(y1_hbm.at[idx_j], yj1, sem),
            pltpu.async_copy(y0_hbm.at[idx_k], yk0, sem),
            pltpu.async_copy(y1_hbm.at[idx_k], yk1, sem),
        ]
        for cp in cps:
            cp.wait()

        def lane_body(l, inner):
            la, va = inner
            s = pl.ds(l * L, L)
            a0, a1 = yi0[s], yi1[s]
            dx = a0 - yj0[s]
            dy = a1 - yj1[s]
            d_ij = 1.0 + (dx * dx + dy * dy)
            ex = a0 - yk0[s]
            ey = a1 - yk1[s]
            d_ik = 1.0 + (ex * ex + ey * ey)
            la = la + w_v[s] * (d_ij / (d_ij + d_ik))
            va = va + jnp.where(d_ij > d_ik, 1.0, 0.0).astype(jnp.float32)
            return la, va

        return lax.fori_loop(0, B // L, lane_body, (loss_acc, viol_acc))

    zero = jnp.zeros((L,), jnp.float32)
    loss_acc, viol_acc = lax.fori_loop(0, NCHUNK, chunk_body, (zero, zero))
    acc_v[...] = loss_acc
    pltpu.sync_copy(acc_v, out_hbm.at[0, pl.ds(wid * L, L)])
    acc_v[...] = viol_acc
    pltpu.sync_copy(acc_v, out_hbm.at[1, pl.ds(wid * L, L)])


@jax.jit
def kernel(Y, weights, triplets):
    y0 = Y[:, 0]
    y1 = Y[:, 1]
    ti = triplets[:, 0].astype(jnp.int32)
    tj = triplets[:, 1].astype(jnp.int32)
    tk = triplets[:, 2].astype(jnp.int32)

    mesh = plsc.VectorSubcoreMesh(core_axis_name="c", subcore_axis_name="s")
    run = pl.kernel(
        _tri_kernel,
        out_type=jax.ShapeDtypeStruct((2, NW * L), jnp.float32),
        mesh=mesh,
        scratch_types=[
            pltpu.VMEM((B,), jnp.int32),
            pltpu.VMEM((B,), jnp.int32),
            pltpu.VMEM((B,), jnp.int32),
            pltpu.VMEM((B,), jnp.float32),
            pltpu.VMEM((B,), jnp.float32),
            pltpu.VMEM((B,), jnp.float32),
            pltpu.VMEM((B,), jnp.float32),
            pltpu.VMEM((B,), jnp.float32),
            pltpu.VMEM((B,), jnp.float32),
            pltpu.VMEM((B,), jnp.float32),
            pltpu.VMEM((L,), jnp.float32),
            pltpu.SemaphoreType.DMA,
        ],
    )
    partials = run(y0, y1, weights, ti, tj, tk)
    loss = jnp.sum(partials[0])
    num_viol = jnp.sum(partials[1])
    return (loss, num_viol)
